# baseline (speedup 1.0000x reference)
import functools

import jax
import jax.numpy as jnp
from jax import lax
from jax.experimental import pallas as pl
from jax.experimental.pallas import tpu as pltpu
from jax.experimental.pallas import tpu_sc as plsc

_B, _L, _D = 4096, 128, 64
_NC, _NS = 2, 16
_NW = _NC * _NS
_BPW = _B // _NW
_NBUF = 4


@functools.partial(
    pl.kernel,
    out_type=jax.ShapeDtypeStruct((_B * _L, _D), jnp.float32),
    mesh=plsc.VectorSubcoreMesh(
        core_axis_name="c", subcore_axis_name="s",
        num_cores=_NC, num_subcores=_NS,
    ),
    scratch_types=[
        pltpu.VMEM((_L,), jnp.int32),
        pltpu.VMEM((_NBUF, _L), jnp.int32),
        pltpu.VMEM((_NBUF, _L, _D), jnp.float32),
        [pltpu.SemaphoreType.DMA] * _NBUF,
        [pltpu.SemaphoreType.DMA] * _NBUF,
    ],
    compiler_params=pltpu.CompilerParams(use_tc_tiling_on_sc=False),
)
def _sc_interleave(x_hbm, p_hbm, out_hbm, p_v, idx_v, rows_v, gsems, ssems):
    wid = lax.axis_index("s") * _NC + lax.axis_index("c")
    b0 = wid * _BPW

    pltpu.sync_copy(p_hbm, p_v)

    def fill_idx(slot, b):
        base = (b0 + b) * _L
        for j in range(_L // 16):
            sl = pl.ds(j * 16, 16)
            idx_v[slot, sl] = p_v[sl] + base

    def gather(slot):
        return pltpu.make_async_copy(
            x_hbm.at[idx_v.at[slot]], rows_v.at[slot], gsems[slot])

    def store(slot, b):
        return pltpu.make_async_copy(
            rows_v.at[slot], out_hbm.at[pl.ds((b0 + b) * _L, _L)],
            ssems[slot])

    ngroups = _BPW // _NBUF

    def group(g, carry):
        base = g * _NBUF
        for j in range(_NBUF):
            @pl.when(g > 0)
            def _():
                store(j, base + j).wait()
            fill_idx(j, base + j)
            gather(j).start()
        for j in range(_NBUF):
            gather(j).wait()
            store(j, base + j).start()
        return carry

    lax.fori_loop(0, ngroups, group, 0)
    last = (ngroups - 1) * _NBUF
    for j in range(_NBUF):
        store(j, last + j).wait()


@functools.partial(
    pl.kernel,
    out_type=jax.ShapeDtypeStruct((_B, _L, _D), jnp.float32),
    mesh=plsc.VectorSubcoreMesh(
        core_axis_name="c", subcore_axis_name="s",
        num_cores=_NC, num_subcores=_NS,
    ),
    scratch_types=[
        pltpu.VMEM((_NBUF, _BPW, _D), jnp.float32),
        [pltpu.SemaphoreType.DMA] * _NBUF,
        [pltpu.SemaphoreType.DMA] * _NBUF,
    ],
    compiler_params=pltpu.CompilerParams(use_tc_tiling_on_sc=False),
)
def _sc_interleave_strided(x_hbm, p_hbm, out_hbm, rows_v, gsems, ssems):
    wid = lax.axis_index("s") * _NC + lax.axis_index("c")
    b0 = wid * _BPW

    def gather(slot, l):
        return pltpu.make_async_copy(
            x_hbm.at[pl.ds(b0, _BPW), _L - 1 - l], rows_v.at[slot],
            gsems[slot])

    def store(slot, l):
        return pltpu.make_async_copy(
            rows_v.at[slot], out_hbm.at[pl.ds(b0, _BPW), l], ssems[slot])

    ngroups = _L // _NBUF

    def group(g, carry):
        base = g * _NBUF
        for j in range(_NBUF):
            @pl.when(g > 0)
            def _():
                store(j, base + j).wait()
            gather(j, base + j).start()
        for j in range(_NBUF):
            gather(j, base + j).wait()
            store(j, base + j).start()
        return carry

    lax.fori_loop(0, ngroups, group, 0)
    last = (ngroups - 1) * _NBUF
    for j in range(_NBUF):
        store(j, last + j).wait()


@functools.partial(
    pl.kernel,
    out_type=jax.ShapeDtypeStruct((_B * _L, _D), jnp.float32),
    mesh=plsc.VectorSubcoreMesh(
        core_axis_name="c", subcore_axis_name="s",
        num_cores=_NC, num_subcores=_NS,
    ),
    scratch_types=[
        pltpu.VMEM((_NBUF, _L, _D), jnp.float32),
        pltpu.VMEM((_NBUF, _L, _D), jnp.float32),
        [pltpu.SemaphoreType.DMA] * _NBUF,
        [pltpu.SemaphoreType.DMA] * _NBUF,
    ],
    compiler_params=pltpu.CompilerParams(use_tc_tiling_on_sc=False),
)
def _sc_interleave_rev(x_hbm, p_hbm, out_hbm, in_v, out_v, gsems, ssems):
    wid = lax.axis_index("s") * _NC + lax.axis_index("c")
    b0 = wid * _BPW

    def gather(slot, b):
        return pltpu.make_async_copy(
            x_hbm.at[pl.ds((b0 + b) * _L, _L)], in_v.at[slot], gsems[slot])

    def store(slot, b):
        return pltpu.make_async_copy(
            out_v.at[slot], out_hbm.at[pl.ds((b0 + b) * _L, _L)],
            ssems[slot])

    def permute(slot):
        @pl.loop(0, _L, unroll=8)
        def _(l):
            src = _L - 1 - l
            for u in range(_D // 16):
                sl = pl.ds(u * 16, 16)
                out_v[slot, l, sl] = in_v[slot, src, sl]

    ngroups = _BPW // _NBUF

    for j in range(_NBUF):
        gather(j, j).start()

    def group(g, carry):
        base = g * _NBUF
        for j in range(_NBUF):
            gather(j, base + j).wait()

            @pl.when(g > 0)
            def _():
                store(j, base + j).wait()
            permute(j)
            store(j, base + j).start()

            @pl.when(g + 1 < ngroups)
            def _():
                gather(j, base + _NBUF + j).start()
        return carry

    lax.fori_loop(0, ngroups, group, 0)
    last = (ngroups - 1) * _NBUF
    for j in range(_NBUF):
        store(j, last + j).wait()


_BT = 16


def _tc_body(p_ref, x_ref, o_ref):
    onehot = jnp.where(
        jax.lax.broadcasted_iota(jnp.int32, (_L, _L), 1) == p_ref[...],
        1.0, 0.0).astype(jnp.float32)
    for b in range(_BT):
        o_ref[b] = jax.lax.dot_general(
            onehot, x_ref[b], (((1,), (0,)), ((), ())),
            preferred_element_type=jnp.float32)


def _tc_interleave(x, p_array):
    nb = x.shape[0] // _BT
    return pl.pallas_call(
        _tc_body,
        grid=(nb,),
        in_specs=[
            pl.BlockSpec((_L, 1), lambda i: (0, 0)),
            pl.BlockSpec((_BT, _L, _D), lambda i: (i, 0, 0)),
        ],
        out_specs=pl.BlockSpec((_BT, _L, _D), lambda i: (i, 0, 0)),
        out_shape=jax.ShapeDtypeStruct(x.shape, jnp.float32),
    )(p_array.reshape(_L, 1), x)


def _tc_copy_body(p_ref, x_ref, o_ref):
    del p_ref
    o_ref[...] = x_ref[...]


def _tc_flip(x, p_array):
    nb = x.shape[0]
    return pl.pallas_call(
        _tc_copy_body,
        grid_spec=pltpu.PrefetchScalarGridSpec(
            num_scalar_prefetch=1,
            grid=(_L,),
            in_specs=[pl.BlockSpec((nb, 1, _D),
                                   lambda i, p: (0, p[i], 0))],
            out_specs=pl.BlockSpec((nb, 1, _D),
                                   lambda i, p: (0, i, 0)),
        ),
        out_shape=jax.ShapeDtypeStruct(x.shape, jnp.float32),
    )(p_array, x)


_BTR = 32
_LD = _L * _D
_NCB = _LD // 128


def _tc_rollflip_body(x_ref, o_ref):
    for k in range(_NCB):
        src = pl.ds((_NCB - 1 - k) * 128, 128)
        dst = pl.ds(k * 128, 128)
        o_ref[:, dst] = pltpu.roll(x_ref[:, src], _D, axis=1)


def _tc_rollflip(x):
    nb = x.shape[0]
    x2 = x.reshape(nb, _LD)
    out = pl.pallas_call(
        _tc_rollflip_body,
        grid=(nb // _BTR,),
        in_specs=[pl.BlockSpec((_BTR, _LD), lambda i: (i, 0))],
        out_specs=pl.BlockSpec((_BTR, _LD), lambda i: (i, 0)),
        out_shape=jax.ShapeDtypeStruct(x2.shape, jnp.float32),
    )(x2)
    return out.reshape(nb, _L, _D)


_NSEM = 8


def _tc_dma_body(p_ref, x_ref, o_ref, sems):
    for l in range(_L):
        pltpu.make_async_copy(
            x_ref.at[:, p_ref[l], :], o_ref.at[:, l, :],
            sems.at[l % _NSEM]).start()
    for l in range(_L):
        pltpu.make_async_copy(
            x_ref.at[:, 0, :], o_ref.at[:, l, :], sems.at[l % _NSEM]).wait()


def _tc_dma_permute(x, p_array):
    return pl.pallas_call(
        _tc_dma_body,
        grid_spec=pltpu.PrefetchScalarGridSpec(
            num_scalar_prefetch=1,
            grid=(),
            in_specs=[pl.BlockSpec(memory_space=pl.ANY)],
            out_specs=pl.BlockSpec(memory_space=pl.ANY),
            scratch_shapes=[pltpu.SemaphoreType.DMA((_NSEM,))],
        ),
        out_shape=jax.ShapeDtypeStruct(x.shape, jnp.float32),
    )(p_array, x)


_RING = 16
_LAG = 8
_NSP = 2
_BH = _B // _NSP


def _tc_mq_body(p_ref, x_ref, o_ref, slabs, gsems, ssems):
    def gather(l, h):
        return pltpu.make_async_copy(
            x_ref.at[pl.ds(h * _BH, _BH), p_ref[l], :],
            slabs.at[l % _RING, h], gsems.at[l % _RING, h])

    def store(l, h):
        return pltpu.make_async_copy(
            slabs.at[l % _RING, h],
            o_ref.at[pl.ds(h * _BH, _BH), l, :], ssems.at[l % _RING, h])

    for l in range(_RING):
        for h in range(_NSP):
            gather(l, h).start()
    for l in range(_L):
        for h in range(_NSP):
            gather(l, h).wait()
            store(l, h).start()
        if l >= _LAG:
            m = l - _LAG
            for h in range(_NSP):
                store(m, h).wait()
                if m + _RING < _L:
                    gather(m + _RING, h).start()
    for l in range(_L - _LAG, _L):
        for h in range(_NSP):
            store(l, h).wait()


def _tc_mq_permute(x, p_array):
    return pl.pallas_call(
        _tc_mq_body,
        grid_spec=pltpu.PrefetchScalarGridSpec(
            num_scalar_prefetch=1,
            grid=(),
            in_specs=[pl.BlockSpec(memory_space=pl.ANY)],
            out_specs=pl.BlockSpec(memory_space=pl.ANY),
            scratch_shapes=[
                pltpu.VMEM((_RING, _NSP, _BH, _D), jnp.float32),
                pltpu.SemaphoreType.DMA((_RING, _NSP)),
                pltpu.SemaphoreType.DMA((_RING, _NSP)),
            ],
        ),
        out_shape=jax.ShapeDtypeStruct(x.shape, jnp.float32),
    )(p_array, x)


def kernel(inputs, p_array):
    return _tc_mq_permute(inputs, p_array)


# speedup vs baseline: 1.1427x; 1.1427x over previous
import jax
import jax.numpy as jnp
from jax.experimental import pallas as pl
from jax.experimental.pallas import tpu as pltpu

_B, _L, _D = 4096, 128, 64


def _copy_body(p_ref, x_ref, o_ref):
    del p_ref
    o_ref[...] = x_ref[...]


def _permute_rows(x, p_array):
    nb = x.shape[0]
    x4 = x.reshape(nb, _L, 1, _D)
    out = pl.pallas_call(
        _copy_body,
        grid_spec=pltpu.PrefetchScalarGridSpec(
            num_scalar_prefetch=1,
            grid=(_L,),
            in_specs=[pl.BlockSpec((nb, 1, 1, _D),
                                   lambda i, p: (0, p[i], 0, 0))],
            out_specs=pl.BlockSpec((nb, 1, 1, _D),
                                   lambda i, p: (0, i, 0, 0)),
        ),
        out_shape=jax.ShapeDtypeStruct(x4.shape, jnp.float32),
    )(p_array, x4)
    return out.reshape(nb, _L, _D)


def kernel(inputs, p_array):
    return _permute_rows(inputs, p_array)
